# baseline (speedup 1.0000x reference)
import functools

import jax
import jax.numpy as jnp
from jax import lax
from jax.experimental import pallas as pl
from jax.experimental.pallas import tpu as pltpu
from jax.experimental.pallas import tpu_sc as plsc

N_ATOMS = 3_200_000
B_SEG = 10_000
NC = 2
NS = 16
NW = NC * NS
PER_W = N_ATOMS // NW
CHUNK = 10_000
NCHUNK = PER_W // CHUNK
VREGS = CHUNK // 16
LANE = 16

_mesh = plsc.VectorSubcoreMesh(
    core_axis_name="c", subcore_axis_name="s", num_cores=NC, num_subcores=NS
)


@functools.partial(
    pl.kernel,
    out_type=jax.ShapeDtypeStruct((NW, B_SEG), jnp.float32),
    mesh=_mesh,
    scratch_types=[
        pltpu.VMEM((CHUNK * 3,), jnp.float32),
        pltpu.VMEM((CHUNK,), jnp.float32),
        pltpu.VMEM((CHUNK,), jnp.int32),
        pltpu.VMEM((B_SEG,), jnp.float32),
    ],
    compiler_params=pltpu.CompilerParams(needs_layout_passes=False),
)
def _sc_energy(pos_hbm, q_hbm, ids_hbm, out_hbm, pbuf, qbuf, ibuf, acc):
    wid = lax.axis_index("c") * NS + lax.axis_index("s")
    base = wid * PER_W

    def _zero(i, _):
        acc[pl.ds(i * LANE, LANE)] = jnp.zeros((LANE,), jnp.float32)
        return _
    lax.fori_loop(0, B_SEG // LANE, _zero, None)

    lanes = lax.iota(jnp.int32, LANE)

    def _chunk(ci, _):
        a0 = base + ci * CHUNK
        pltpu.sync_copy(pos_hbm.at[pl.ds(a0 * 3, CHUNK * 3)], pbuf)
        pltpu.sync_copy(q_hbm.at[pl.ds(a0, CHUNK)], qbuf)
        pltpu.sync_copy(ids_hbm.at[pl.ds(a0, CHUNK)], ibuf)

        def _vec(i, _):
            pi = i * (3 * LANE) + lanes * 3
            x = plsc.load_gather(pbuf, [pi])
            y = plsc.load_gather(pbuf, [pi + 1])
            z = plsc.load_gather(pbuf, [pi + 2])
            q = qbuf[pl.ds(i * LANE, LANE)]
            ids = ibuf[pl.ds(i * LANE, LANE)]
            n2 = jnp.maximum(x * x + y * y + z * z, jnp.float32(1e-37))
            r = plsc.bitcast(
                jnp.int32(0x5F3759DF) - (plsc.bitcast(n2, jnp.int32) >> 1),
                jnp.float32,
            )
            h = jnp.float32(0.5) * n2
            for _i in range(3):
                r = r * (jnp.float32(1.5) - h * r * r)
            e = q * (n2 * r)
            plsc.addupdate_scatter(acc, [ids], e)
            return _

        lax.fori_loop(0, VREGS, _vec, None)
        return _

    lax.fori_loop(0, NCHUNK, _chunk, None)
    pltpu.sync_copy(acc, out_hbm.at[wid])


def _combine_body(p_ref, o_ref):
    o_ref[...] = jnp.sum(p_ref[...], axis=0, keepdims=True)


_combine = pl.pallas_call(
    _combine_body,
    out_shape=jax.ShapeDtypeStruct((1, B_SEG), jnp.float32),
)


def kernel(positions, node_charges, batch_idx, B):
    del B
    partials = _sc_energy(positions.reshape(-1), node_charges, batch_idx)
    return _combine(partials).reshape(B_SEG, 1)


# speedup vs baseline: 10.9213x; 10.9213x over previous
import functools

import jax
import jax.numpy as jnp
from jax import lax
from jax.experimental import pallas as pl
from jax.experimental.pallas import tpu as pltpu
from jax.experimental.pallas import tpu_sc as plsc

N_ATOMS = 3_200_000
B_SEG = 10_000
NC = 2
NS = 16
NW = NC * NS
PER_W = N_ATOMS // NW
CHUNK = 10_000
NCHUNK = PER_W // CHUNK
VREGS = CHUNK // 16
LANE = 16

_mesh = plsc.VectorSubcoreMesh(
    core_axis_name="c", subcore_axis_name="s", num_cores=NC, num_subcores=NS
)


@functools.partial(
    pl.kernel,
    out_type=jax.ShapeDtypeStruct((NW, B_SEG), jnp.float32),
    mesh=_mesh,
    scratch_types=[
        pltpu.VMEM((CHUNK,), jnp.float32),
        pltpu.VMEM((CHUNK,), jnp.float32),
        pltpu.VMEM((CHUNK,), jnp.float32),
        pltpu.VMEM((CHUNK,), jnp.float32),
        pltpu.VMEM((CHUNK,), jnp.int32),
        pltpu.VMEM((B_SEG,), jnp.float32),
    ],
    compiler_params=pltpu.CompilerParams(
        needs_layout_passes=False, use_tc_tiling_on_sc=False
    ),
)
def _sc_energy(pos_hbm, q_hbm, ids_hbm, out_hbm, xbuf, ybuf, zbuf, qbuf, ibuf, acc):
    wid = lax.axis_index("c") * NS + lax.axis_index("s")
    base = wid * PER_W

    def _zero(i, _):
        acc[pl.ds(i * LANE, LANE)] = jnp.zeros((LANE,), jnp.float32)
        return _
    lax.fori_loop(0, B_SEG // LANE, _zero, None)

    def _chunk(ci, _):
        a0 = base + ci * CHUNK
        pltpu.sync_copy(pos_hbm.at[0, pl.ds(a0, CHUNK)], xbuf)
        pltpu.sync_copy(pos_hbm.at[1, pl.ds(a0, CHUNK)], ybuf)
        pltpu.sync_copy(pos_hbm.at[2, pl.ds(a0, CHUNK)], zbuf)
        pltpu.sync_copy(q_hbm.at[pl.ds(a0, CHUNK)], qbuf)
        pltpu.sync_copy(ids_hbm.at[pl.ds(a0, CHUNK)], ibuf)

        def _vec(i, _):
            x = xbuf[pl.ds(i * LANE, LANE)]
            y = ybuf[pl.ds(i * LANE, LANE)]
            z = zbuf[pl.ds(i * LANE, LANE)]
            q = qbuf[pl.ds(i * LANE, LANE)]
            ids = ibuf[pl.ds(i * LANE, LANE)]
            n2 = jnp.maximum(x * x + y * y + z * z, jnp.float32(1e-37))
            r = plsc.bitcast(
                jnp.int32(0x5F3759DF) - (plsc.bitcast(n2, jnp.int32) >> 1),
                jnp.float32,
            )
            h = jnp.float32(0.5) * n2
            for _i in range(3):
                r = r * (jnp.float32(1.5) - h * r * r)
            e = q * (n2 * r)
            plsc.addupdate_scatter(acc, [ids], e)
            return _

        lax.fori_loop(0, VREGS, _vec, None)
        return _

    lax.fori_loop(0, NCHUNK, _chunk, None)
    pltpu.sync_copy(acc, out_hbm.at[wid])


def _combine_body(p_ref, o_ref):
    o_ref[...] = jnp.sum(p_ref[...], axis=0, keepdims=True)


_combine = pl.pallas_call(
    _combine_body,
    out_shape=jax.ShapeDtypeStruct((1, B_SEG), jnp.float32),
)


def kernel(positions, node_charges, batch_idx, B):
    del B
    pos_t = jnp.transpose(positions)
    partials = _sc_energy(pos_t, node_charges, batch_idx)
    return _combine(partials).reshape(B_SEG, 1)


# speedup vs baseline: 26.1016x; 2.3900x over previous
import functools

import jax
import jax.numpy as jnp
from jax import lax
from jax.experimental import pallas as pl
from jax.experimental.pallas import tpu as pltpu
from jax.experimental.pallas import tpu_sc as plsc

N_ATOMS = 3_200_000
B_SEG = 10_000
NC = 2
NS = 16
NW = NC * NS
PER_W = N_ATOMS // NW
CHUNK = 10_000
NCHUNK = PER_W // CHUNK
VREGS = CHUNK // 16
LANE = 16

_mesh = plsc.VectorSubcoreMesh(
    core_axis_name="c", subcore_axis_name="s", num_cores=NC, num_subcores=NS
)


@functools.partial(
    pl.kernel,
    out_type=jax.ShapeDtypeStruct((NW, B_SEG), jnp.float32),
    mesh=_mesh,
    scratch_types=[
        pltpu.VMEM((CHUNK,), jnp.float32),
        pltpu.VMEM((CHUNK,), jnp.float32),
        pltpu.VMEM((CHUNK,), jnp.float32),
        pltpu.VMEM((CHUNK,), jnp.float32),
        pltpu.VMEM((CHUNK,), jnp.int32),
        pltpu.VMEM((B_SEG,), jnp.float32),
    ],
    compiler_params=pltpu.CompilerParams(
        needs_layout_passes=False, use_tc_tiling_on_sc=False
    ),
)
def _sc_energy(x_hbm, y_hbm, z_hbm, q_hbm, ids_hbm, out_hbm, xbuf, ybuf, zbuf, qbuf, ibuf, acc):
    wid = lax.axis_index("c") * NS + lax.axis_index("s")
    base = wid * PER_W

    def _zero(i, _):
        acc[pl.ds(i * LANE, LANE)] = jnp.zeros((LANE,), jnp.float32)
        return _
    lax.fori_loop(0, B_SEG // LANE, _zero, None)

    def _chunk(ci, _):
        a0 = base + ci * CHUNK
        pltpu.sync_copy(x_hbm.at[pl.ds(a0, CHUNK)], xbuf)
        pltpu.sync_copy(y_hbm.at[pl.ds(a0, CHUNK)], ybuf)
        pltpu.sync_copy(z_hbm.at[pl.ds(a0, CHUNK)], zbuf)
        pltpu.sync_copy(q_hbm.at[pl.ds(a0, CHUNK)], qbuf)
        pltpu.sync_copy(ids_hbm.at[pl.ds(a0, CHUNK)], ibuf)

        def _vec(i, _):
            x = xbuf[pl.ds(i * LANE, LANE)]
            y = ybuf[pl.ds(i * LANE, LANE)]
            z = zbuf[pl.ds(i * LANE, LANE)]
            q = qbuf[pl.ds(i * LANE, LANE)]
            ids = ibuf[pl.ds(i * LANE, LANE)]
            n2 = jnp.maximum(x * x + y * y + z * z, jnp.float32(1e-37))
            r = plsc.bitcast(
                jnp.int32(0x5F3759DF) - (plsc.bitcast(n2, jnp.int32) >> 1),
                jnp.float32,
            )
            h = jnp.float32(0.5) * n2
            for _i in range(3):
                r = r * (jnp.float32(1.5) - h * r * r)
            e = q * (n2 * r)
            plsc.addupdate_scatter(acc, [ids], e)
            return _

        lax.fori_loop(0, VREGS, _vec, None)
        return _

    lax.fori_loop(0, NCHUNK, _chunk, None)
    pltpu.sync_copy(acc, out_hbm.at[wid])


def _combine_body(p_ref, o_ref):
    o_ref[...] = jnp.sum(p_ref[...], axis=0, keepdims=True)


_combine = pl.pallas_call(
    _combine_body,
    out_shape=jax.ShapeDtypeStruct((1, B_SEG), jnp.float32),
)


def kernel(positions, node_charges, batch_idx, B):
    del B
    partials = _sc_energy(
        positions[:, 0], positions[:, 1], positions[:, 2],
        node_charges, batch_idx,
    )
    return _combine(partials).reshape(B_SEG, 1)


# speedup vs baseline: 32.3566x; 1.2396x over previous
import functools

import jax
import jax.numpy as jnp
from jax import lax
from jax.experimental import pallas as pl
from jax.experimental.pallas import tpu as pltpu
from jax.experimental.pallas import tpu_sc as plsc

N_ATOMS = 3_200_000
B_SEG = 10_000
NC = 2
NS = 16
NW = NC * NS
PER_W = N_ATOMS // NW
CHUNK = 10_000
NCHUNK = PER_W // CHUNK
LANE = 16
UNROLL = 5
NITER = CHUNK // (LANE * UNROLL)

_mesh = plsc.VectorSubcoreMesh(
    core_axis_name="c", subcore_axis_name="s", num_cores=NC, num_subcores=NS
)

_chunk_f32 = pltpu.VMEM((CHUNK,), jnp.float32)
_chunk_i32 = pltpu.VMEM((CHUNK,), jnp.int32)


@functools.partial(
    pl.kernel,
    out_type=jax.ShapeDtypeStruct((NW, B_SEG), jnp.float32),
    mesh=_mesh,
    scratch_types=[
        _chunk_f32, _chunk_f32, _chunk_f32, _chunk_f32, _chunk_i32,
        _chunk_f32, _chunk_f32, _chunk_f32, _chunk_f32, _chunk_i32,
        pltpu.VMEM((B_SEG,), jnp.float32),
        pltpu.SemaphoreType.DMA,
        pltpu.SemaphoreType.DMA,
    ],
    compiler_params=pltpu.CompilerParams(
        needs_layout_passes=False, use_tc_tiling_on_sc=False
    ),
)
def _sc_energy(
    x_hbm, y_hbm, z_hbm, q_hbm, ids_hbm, out_hbm,
    xb0, yb0, zb0, qb0, ib0,
    xb1, yb1, zb1, qb1, ib1,
    acc, sem0, sem1,
):
    wid = lax.axis_index("c") * NS + lax.axis_index("s")
    base = wid * PER_W
    bufs = ((xb0, yb0, zb0, qb0, ib0, sem0), (xb1, yb1, zb1, qb1, ib1, sem1))

    def _start(ci, bset):
        xb, yb, zb, qb, ib, sem = bset
        a0 = base + ci * CHUNK
        return [
            pltpu.async_copy(x_hbm.at[pl.ds(a0, CHUNK)], xb, sem),
            pltpu.async_copy(y_hbm.at[pl.ds(a0, CHUNK)], yb, sem),
            pltpu.async_copy(z_hbm.at[pl.ds(a0, CHUNK)], zb, sem),
            pltpu.async_copy(q_hbm.at[pl.ds(a0, CHUNK)], qb, sem),
            pltpu.async_copy(ids_hbm.at[pl.ds(a0, CHUNK)], ib, sem),
        ]

    descs = [_start(0, bufs[0]), None]

    def _zero(i, _):
        acc[pl.ds(i * LANE, LANE)] = jnp.zeros((LANE,), jnp.float32)
        return _
    lax.fori_loop(0, B_SEG // LANE, _zero, None)

    def _compute(bset):
        xb, yb, zb, qb, ib, _sem = bset

        def _vec(i, _):
            for u in range(UNROLL):
                o = (i * UNROLL + u) * LANE
                x = xb[pl.ds(o, LANE)]
                y = yb[pl.ds(o, LANE)]
                z = zb[pl.ds(o, LANE)]
                q = qb[pl.ds(o, LANE)]
                ids = ib[pl.ds(o, LANE)]
                n2 = jnp.maximum(x * x + y * y + z * z, jnp.float32(1e-37))
                r = plsc.bitcast(
                    jnp.int32(0x5F3759DF) - (plsc.bitcast(n2, jnp.int32) >> 1),
                    jnp.float32,
                )
                h = jnp.float32(0.5) * n2
                for _i in range(2):
                    r = r * (jnp.float32(1.5) - h * r * r)
                e = q * (n2 * r)
                plsc.addupdate_scatter(acc, [ids], e)
            return _

        lax.fori_loop(0, NITER, _vec, None)

    for ci in range(NCHUNK):
        if ci + 1 < NCHUNK:
            descs[(ci + 1) % 2] = _start(ci + 1, bufs[(ci + 1) % 2])
        for d in descs[ci % 2]:
            d.wait()
        _compute(bufs[ci % 2])

    pltpu.sync_copy(acc, out_hbm.at[wid])


def _combine_body(p_ref, o_ref):
    o_ref[...] = jnp.sum(p_ref[...], axis=0, keepdims=True)


_combine = pl.pallas_call(
    _combine_body,
    out_shape=jax.ShapeDtypeStruct((1, B_SEG), jnp.float32),
)


def kernel(positions, node_charges, batch_idx, B):
    del B
    partials = _sc_energy(
        positions[:, 0], positions[:, 1], positions[:, 2],
        node_charges, batch_idx,
    )
    return _combine(partials).reshape(B_SEG, 1)


# speedup vs baseline: 50.4560x; 1.5594x over previous
import functools

import jax
import jax.numpy as jnp
from jax import lax
from jax.experimental import pallas as pl
from jax.experimental.pallas import tpu as pltpu
from jax.experimental.pallas import tpu_sc as plsc

N_ATOMS = 3_200_000
B_SEG = 10_000
NC = 2
NS = 16
NW = NC * NS
PER_W = N_ATOMS // NW
CHUNK = 10_000
NCHUNK = PER_W // CHUNK
LANE = 16
UNROLL = 5
NITER = CHUNK // (LANE * UNROLL)

_mesh = plsc.VectorSubcoreMesh(
    core_axis_name="c", subcore_axis_name="s", num_cores=NC, num_subcores=NS
)

_chunk_f32 = pltpu.VMEM((CHUNK,), jnp.float32)
_chunk_i32 = pltpu.VMEM((CHUNK,), jnp.int32)


@functools.partial(
    pl.kernel,
    out_type=jax.ShapeDtypeStruct((NW, B_SEG), jnp.float32),
    mesh=_mesh,
    scratch_types=[
        _chunk_f32, _chunk_f32, _chunk_f32, _chunk_f32, _chunk_i32,
        _chunk_f32, _chunk_f32, _chunk_f32, _chunk_f32, _chunk_i32,
        pltpu.VMEM((B_SEG,), jnp.float32),
        pltpu.SemaphoreType.DMA,
        pltpu.SemaphoreType.DMA,
    ],
    compiler_params=pltpu.CompilerParams(
        needs_layout_passes=False, use_tc_tiling_on_sc=False
    ),
)
def _sc_energy(
    x_hbm, y_hbm, z_hbm, q_hbm, ids_hbm, out_hbm,
    xb0, yb0, zb0, qb0, ib0,
    xb1, yb1, zb1, qb1, ib1,
    acc, sem0, sem1,
):
    wid = lax.axis_index("c") * NS + lax.axis_index("s")
    base = wid * PER_W
    bufs = ((xb0, yb0, zb0, qb0, ib0, sem0), (xb1, yb1, zb1, qb1, ib1, sem1))

    def _start(ci, bset):
        xb, yb, zb, qb, ib, sem = bset
        a0 = base + ci * CHUNK
        return [
            pltpu.async_copy(x_hbm.at[pl.ds(a0, CHUNK)], xb, sem),
            pltpu.async_copy(y_hbm.at[pl.ds(a0, CHUNK)], yb, sem),
            pltpu.async_copy(z_hbm.at[pl.ds(a0, CHUNK)], zb, sem),
            pltpu.async_copy(q_hbm.at[pl.ds(a0, CHUNK)], qb, sem),
            pltpu.async_copy(ids_hbm.at[pl.ds(a0, CHUNK)], ib, sem),
        ]

    descs = [_start(0, bufs[0]), None]

    def _zero(i, _):
        acc[pl.ds(i * LANE, LANE)] = jnp.zeros((LANE,), jnp.float32)
        return _
    lax.fori_loop(0, B_SEG // LANE, _zero, None)

    def _compute(bset):
        xb, yb, zb, qb, ib, _sem = bset

        @plsc.parallel_loop(0, CHUNK // LANE, unroll=UNROLL)
        def _vec(i):
            o = i * LANE
            x = xb[pl.ds(o, LANE)]
            y = yb[pl.ds(o, LANE)]
            z = zb[pl.ds(o, LANE)]
            q = qb[pl.ds(o, LANE)]
            ids = ib[pl.ds(o, LANE)]
            n2 = jnp.maximum(x * x + y * y + z * z, jnp.float32(1e-37))
            r = plsc.bitcast(
                jnp.int32(0x5F3759DF) - (plsc.bitcast(n2, jnp.int32) >> 1),
                jnp.float32,
            )
            h = jnp.float32(0.5) * n2
            for _i in range(2):
                r = r * (jnp.float32(1.5) - h * r * r)
            e = q * (n2 * r)
            plsc.addupdate_scatter(acc, [ids], e)

    for ci in range(NCHUNK):
        if ci + 1 < NCHUNK:
            descs[(ci + 1) % 2] = _start(ci + 1, bufs[(ci + 1) % 2])
        for d in descs[ci % 2]:
            d.wait()
        _compute(bufs[ci % 2])

    pltpu.sync_copy(acc, out_hbm.at[wid])


def _combine_body(p_ref, o_ref):
    o_ref[...] = jnp.sum(p_ref[...], axis=0, keepdims=True)


_combine = pl.pallas_call(
    _combine_body,
    out_shape=jax.ShapeDtypeStruct((1, B_SEG), jnp.float32),
)


def kernel(positions, node_charges, batch_idx, B):
    del B
    partials = _sc_energy(
        positions[:, 0], positions[:, 1], positions[:, 2],
        node_charges, batch_idx,
    )
    return _combine(partials).reshape(B_SEG, 1)


# speedup vs baseline: 89.3407x; 1.7707x over previous
import functools

import jax
import jax.numpy as jnp
from jax import lax
from jax.experimental import pallas as pl
from jax.experimental.pallas import tpu as pltpu
from jax.experimental.pallas import tpu_sc as plsc

N_ATOMS = 3_200_000
B_SEG = 10_000
NC = 2
NS = 16
NW = NC * NS
PER_W = N_ATOMS // NW
CHUNK = 10_000
NCHUNK = PER_W // CHUNK
LANE = 16
UNROLL = 5
NITER = CHUNK // (LANE * UNROLL)

_mesh = plsc.VectorSubcoreMesh(
    core_axis_name="c", subcore_axis_name="s", num_cores=NC, num_subcores=NS
)

_chunk_f32 = pltpu.VMEM((CHUNK,), jnp.float32)
_ids_i32 = pltpu.VMEM((CHUNK + LANE,), jnp.int32)


@functools.partial(
    pl.kernel,
    out_type=jax.ShapeDtypeStruct((NW, B_SEG), jnp.float32),
    mesh=_mesh,
    scratch_types=[
        _chunk_f32, _chunk_f32, _chunk_f32, _chunk_f32, _ids_i32,
        _chunk_f32, _chunk_f32, _chunk_f32, _chunk_f32, _ids_i32,
        pltpu.VMEM((B_SEG,), jnp.float32),
        pltpu.SemaphoreType.DMA,
        pltpu.SemaphoreType.DMA,
    ],
    compiler_params=pltpu.CompilerParams(
        needs_layout_passes=False, use_tc_tiling_on_sc=False
    ),
)
def _sc_energy(
    x_hbm, y_hbm, z_hbm, q_hbm, ids_hbm, out_hbm,
    xb0, yb0, zb0, qb0, ib0,
    xb1, yb1, zb1, qb1, ib1,
    acc, sem0, sem1,
):
    wid = lax.axis_index("c") * NS + lax.axis_index("s")
    base = wid * PER_W
    bufs = ((xb0, yb0, zb0, qb0, ib0, sem0), (xb1, yb1, zb1, qb1, ib1, sem1))

    def _start(ci, bset):
        xb, yb, zb, qb, ib, sem = bset
        a0 = base + ci * CHUNK
        return [
            pltpu.async_copy(x_hbm.at[pl.ds(a0, CHUNK)], xb, sem),
            pltpu.async_copy(y_hbm.at[pl.ds(a0, CHUNK)], yb, sem),
            pltpu.async_copy(z_hbm.at[pl.ds(a0, CHUNK)], zb, sem),
            pltpu.async_copy(q_hbm.at[pl.ds(a0, CHUNK)], qb, sem),
            pltpu.async_copy(ids_hbm.at[pl.ds(a0, CHUNK)], ib.at[pl.ds(0, CHUNK)], sem),
        ]

    descs = [_start(0, bufs[0]), None]

    def _zero(i, _):
        acc[pl.ds(i * LANE, LANE)] = jnp.zeros((LANE,), jnp.float32)
        return _
    lax.fori_loop(0, B_SEG // LANE, _zero, None)
    ib0[pl.ds(CHUNK, LANE)] = jnp.zeros((LANE,), jnp.int32)
    ib1[pl.ds(CHUNK, LANE)] = jnp.zeros((LANE,), jnp.int32)

    lanes = lax.iota(jnp.int32, LANE)
    is_last = lanes == jnp.int32(LANE - 1)
    not_last = lanes != jnp.int32(LANE - 1)

    def _compute(bset):
        xb, yb, zb, qb, ib, _sem = bset

        @plsc.parallel_loop(0, CHUNK // LANE, unroll=UNROLL)
        def _vec(i):
            o = i * LANE
            x = xb[pl.ds(o, LANE)]
            y = yb[pl.ds(o, LANE)]
            z = zb[pl.ds(o, LANE)]
            q = qb[pl.ds(o, LANE)]
            ids = ib[pl.ds(o, LANE)]
            n2 = jnp.maximum(x * x + y * y + z * z, jnp.float32(1e-37))
            r = plsc.bitcast(
                jnp.int32(0x5F3759DF) - (plsc.bitcast(n2, jnp.int32) >> 1),
                jnp.float32,
            )
            h = jnp.float32(0.5) * n2
            for _i in range(2):
                r = r * (jnp.float32(1.5) - h * r * r)
            e = q * (n2 * r)
            c = plsc.cumsum(e)
            nxt = plsc.load_gather(ib, [o + 1 + lanes])
            d = ids != nxt
            plsc.addupdate_scatter(acc, [ids], c, mask=d | is_last)
            plsc.addupdate_scatter(acc, [nxt], -c, mask=d & not_last)

    for ci in range(NCHUNK):
        if ci + 1 < NCHUNK:
            descs[(ci + 1) % 2] = _start(ci + 1, bufs[(ci + 1) % 2])
        for d in descs[ci % 2]:
            d.wait()
        _compute(bufs[ci % 2])

    pltpu.sync_copy(acc, out_hbm.at[wid])


def _combine_body(p_ref, o_ref):
    o_ref[...] = jnp.sum(p_ref[...], axis=0, keepdims=True)


_combine = pl.pallas_call(
    _combine_body,
    out_shape=jax.ShapeDtypeStruct((1, B_SEG), jnp.float32),
)


def kernel(positions, node_charges, batch_idx, B):
    del B
    partials = _sc_energy(
        positions[:, 0], positions[:, 1], positions[:, 2],
        node_charges, batch_idx,
    )
    return _combine(partials).reshape(B_SEG, 1)
